# baseline (speedup 1.0000x reference)
import functools

import jax
import jax.numpy as jnp
from jax import lax
from jax.experimental import pallas as pl
from jax.experimental.pallas import tpu as pltpu
from jax.experimental.pallas import tpu_sc as plsc

BATCH = 16384
DIM = 2048
L = 16
NC = 2
NS = 16
NW = NC * NS
ROWS_PER_W = BATCH // NW
CHUNK_R = 2
NCHUNK = ROWS_PER_W // CHUNK_R
NBUF = 8
assert NCHUNK % NBUF == 0


def _sc_body(z_hbm, idx_hbm, out_hbm, idx_v, in_bufs, out_bufs, sems_i, sems_o):
    wid = lax.axis_index("s") * NC + lax.axis_index("c")
    row0 = wid * ROWS_PER_W

    def in_slice(c):
        return z_hbm.at[pl.ds(row0 + c * CHUNK_R, CHUNK_R)]

    def out_slice(c):
        return out_hbm.at[pl.ds(row0 + c * CHUNK_R, CHUNK_R)]

    def compute(in_v, out_v):
        @plsc.parallel_loop(0, DIM // L, unroll=8)
        def col_body(k):
            colv = idx_v[pl.ds(k * L, L)]
            for r in range(CHUNK_R):
                rsplat = jnp.full((L,), r, jnp.int32)
                vals = plsc.load_gather(in_v, [rsplat, colv])
                out_v[r, pl.ds(k * L, L)] = vals

    for b in range(NBUF):
        pltpu.async_copy(in_slice(b), in_bufs[b], sems_i[b])
    pltpu.sync_copy(idx_hbm, idx_v)

    def ring_body(p, carry):
        for b in range(NBUF):
            c = NBUF * p + b
            pltpu.make_async_copy(in_slice(0), in_bufs[b], sems_i[b]).wait()

            @pl.when(p > 0)
            def _wait_out():
                pltpu.make_async_copy(out_bufs[b], out_slice(0), sems_o[b]).wait()

            compute(in_bufs[b], out_bufs[b])
            pltpu.async_copy(out_bufs[b], out_slice(c), sems_o[b])

            @pl.when(c + NBUF < NCHUNK)
            def _prefetch():
                pltpu.async_copy(in_slice(c + NBUF), in_bufs[b], sems_i[b])

        return carry

    lax.fori_loop(0, NCHUNK // NBUF, ring_body, 0)

    for b in range(NBUF):
        pltpu.make_async_copy(out_bufs[b], out_slice(0), sems_o[b]).wait()


_sc_kernel = functools.partial(
    pl.kernel,
    mesh=plsc.VectorSubcoreMesh(core_axis_name="c", subcore_axis_name="s"),
    out_type=jax.ShapeDtypeStruct((BATCH, DIM), jnp.float32),
    compiler_params=pltpu.CompilerParams(needs_layout_passes=False),
    scratch_types=[
        pltpu.VMEM((DIM,), jnp.int32),
        [pltpu.VMEM((CHUNK_R, DIM), jnp.float32) for _ in range(NBUF)],
        [pltpu.VMEM((CHUNK_R, DIM), jnp.float32) for _ in range(NBUF)],
        [pltpu.SemaphoreType.DMA for _ in range(NBUF)],
        [pltpu.SemaphoreType.DMA for _ in range(NBUF)],
    ],
)(_sc_body)


def kernel(z, perm_indices):
    return _sc_kernel(z, perm_indices)


# speedup vs baseline: 1.0047x; 1.0047x over previous
import functools

import jax
import jax.numpy as jnp
from jax import lax
from jax.experimental import pallas as pl
from jax.experimental.pallas import tpu as pltpu
from jax.experimental.pallas import tpu_sc as plsc

BATCH = 16384
DIM = 2048
L = 16
NC = 2
NS = 16
NW = NC * NS
ROWS_PER_W = BATCH // NW
CHUNK_R = 8
NCHUNK = ROWS_PER_W // CHUNK_R
NBUF_I = 4
NBUF_O = 2
assert NCHUNK % NBUF_I == 0


def _sc_body(z_hbm, idx_hbm, out_hbm, idx_v, in_bufs, out_bufs, sems_i, sems_o):
    wid = lax.axis_index("s") * NC + lax.axis_index("c")
    row0 = wid * ROWS_PER_W

    def in_slice(c):
        return z_hbm.at[pl.ds(row0 + c * CHUNK_R, CHUNK_R)]

    def out_slice(c):
        return out_hbm.at[pl.ds(row0 + c * CHUNK_R, CHUNK_R)]

    def compute(in_v, out_v):
        @plsc.parallel_loop(0, DIM // L, unroll=8)
        def col_body(k):
            colv = idx_v[pl.ds(k * L, L)]
            for r in range(CHUNK_R):
                rsplat = jnp.full((L,), r, jnp.int32)
                vals = plsc.load_gather(in_v, [rsplat, colv])
                out_v[r, pl.ds(k * L, L)] = vals

    for b in range(NBUF_I):
        pltpu.async_copy(in_slice(b), in_bufs[b], sems_i[b])
    pltpu.sync_copy(idx_hbm, idx_v)

    def ring_body(p, carry):
        for b in range(NBUF_I):
            c = NBUF_I * p + b
            bo = b % NBUF_O
            pltpu.make_async_copy(in_slice(0), in_bufs[b], sems_i[b]).wait()

            if b < NBUF_O:

                @pl.when(p > 0)
                def _wait_out():
                    pltpu.make_async_copy(
                        out_bufs[bo], out_slice(0), sems_o[bo]).wait()

            else:
                pltpu.make_async_copy(
                    out_bufs[bo], out_slice(0), sems_o[bo]).wait()

            compute(in_bufs[b], out_bufs[bo])
            pltpu.async_copy(out_bufs[bo], out_slice(c), sems_o[bo])

            @pl.when(c + NBUF_I < NCHUNK)
            def _prefetch():
                pltpu.async_copy(in_slice(c + NBUF_I), in_bufs[b], sems_i[b])

        return carry

    lax.fori_loop(0, NCHUNK // NBUF_I, ring_body, 0)

    for bo in range(NBUF_O):
        pltpu.make_async_copy(out_bufs[bo], out_slice(0), sems_o[bo]).wait()


_sc_kernel = functools.partial(
    pl.kernel,
    mesh=plsc.VectorSubcoreMesh(core_axis_name="c", subcore_axis_name="s"),
    out_type=jax.ShapeDtypeStruct((BATCH, DIM), jnp.float32),
    compiler_params=pltpu.CompilerParams(needs_layout_passes=False),
    scratch_types=[
        pltpu.VMEM((DIM,), jnp.int32),
        [pltpu.VMEM((CHUNK_R, DIM), jnp.float32) for _ in range(NBUF_I)],
        [pltpu.VMEM((CHUNK_R, DIM), jnp.float32) for _ in range(NBUF_O)],
        [pltpu.SemaphoreType.DMA for _ in range(NBUF_I)],
        [pltpu.SemaphoreType.DMA for _ in range(NBUF_O)],
    ],
)(_sc_body)


def kernel(z, perm_indices):
    return _sc_kernel(z, perm_indices)


# speedup vs baseline: 1.0057x; 1.0010x over previous
import functools

import jax
import jax.numpy as jnp
from jax import lax
from jax.experimental import pallas as pl
from jax.experimental.pallas import tpu as pltpu
from jax.experimental.pallas import tpu_sc as plsc

BATCH = 16384
DIM = 2048
L = 16
NC = 2
NS = 16
NW = NC * NS
ROWS_PER_W = BATCH // NW
CHUNK_R = 4
NCHUNK = ROWS_PER_W // CHUNK_R
NBUF = 4
assert NCHUNK % NBUF == 0


def _sc_body(z_hbm, idx_hbm, out_hbm, idx_v, in_bufs, out_bufs, sems_i, sems_o):
    wid = lax.axis_index("s") * NC + lax.axis_index("c")
    row0 = wid * ROWS_PER_W

    def in_slice(c):
        return z_hbm.at[pl.ds(row0 + c * CHUNK_R, CHUNK_R)]

    def out_slice(c):
        return out_hbm.at[pl.ds(row0 + c * CHUNK_R, CHUNK_R)]

    def compute(in_v, out_v):
        @plsc.parallel_loop(0, DIM // L, unroll=8)
        def col_body(k):
            colv = idx_v[pl.ds(k * L, L)]
            for r in range(CHUNK_R):
                rsplat = jnp.full((L,), r, jnp.int32)
                vals = plsc.load_gather(in_v, [rsplat, colv])
                out_v[r, pl.ds(k * L, L)] = vals

    for b in range(NBUF):
        pltpu.async_copy(in_slice(b), in_bufs[b], sems_i[b])
    pltpu.sync_copy(idx_hbm, idx_v)

    def ring_body(p, carry):
        for b in range(NBUF):
            c = NBUF * p + b
            pltpu.make_async_copy(in_slice(0), in_bufs[b], sems_i[b]).wait()

            @pl.when(p > 0)
            def _wait_out():
                pltpu.make_async_copy(out_bufs[b], out_slice(0), sems_o[b]).wait()

            compute(in_bufs[b], out_bufs[b])
            pltpu.async_copy(out_bufs[b], out_slice(c), sems_o[b])

            @pl.when(c + NBUF < NCHUNK)
            def _prefetch():
                pltpu.async_copy(in_slice(c + NBUF), in_bufs[b], sems_i[b])

        return carry

    lax.fori_loop(0, NCHUNK // NBUF, ring_body, 0)

    for b in range(NBUF):
        pltpu.make_async_copy(out_bufs[b], out_slice(0), sems_o[b]).wait()


_sc_kernel = functools.partial(
    pl.kernel,
    mesh=plsc.VectorSubcoreMesh(core_axis_name="c", subcore_axis_name="s"),
    out_type=jax.ShapeDtypeStruct((BATCH, DIM), jnp.float32),
    compiler_params=pltpu.CompilerParams(needs_layout_passes=False),
    scratch_types=[
        pltpu.VMEM((DIM,), jnp.int32),
        [pltpu.VMEM((CHUNK_R, DIM), jnp.float32) for _ in range(NBUF)],
        [pltpu.VMEM((CHUNK_R, DIM), jnp.float32) for _ in range(NBUF)],
        [pltpu.SemaphoreType.DMA for _ in range(NBUF)],
        [pltpu.SemaphoreType.DMA for _ in range(NBUF)],
    ],
)(_sc_body)


def kernel(z, perm_indices):
    return _sc_kernel(z, perm_indices)
